# baseline (speedup 1.0000x reference)
import functools

import jax
import jax.numpy as jnp
from jax import lax
from jax.experimental import pallas as pl
from jax.experimental.pallas import tpu as pltpu
from jax.experimental.pallas import tpu_sc as plsc

N = 10000
E = 320000
NFEAT = 128
HID = 16
NHEADS = 8
NCLASS = 10
SMOO = 0.5

C = 128
NTILE = 16
NSC = 2
RPT = 624
TAILN = N - RPT * NTILE
DPAD = 16

KB = 160
EPB = NTILE * KB * C
KD = 80
EPD = NSC * NTILE * KD * C

_MESH = plsc.VectorSubcoreMesh(core_axis_name="c", subcore_axis_name="s")


def _scale_rows(rows, b, wrow, k, nfeat):
    def group_body(g, carry):
        e0 = g * 16
        wv = wrow[k, pl.ds(e0, 16)]
        for j in range(16):
            w = wv[j]
            for f in range(nfeat // 16):
                sl = pl.ds(f * 16, 16)
                rows[b, e0 + j, sl] = rows[b, e0 + j, sl] * w
        return carry

    lax.fori_loop(0, C // 16, group_body, 0)


SK = 40


def _preload_block(srcs, dsts, ws, srcb, dstb, wb, isem):
    pltpu.async_copy(srcs, srcb, isem)
    pltpu.async_copy(dsts, dstb, isem)
    pltpu.async_copy(ws, wb, isem)
    pltpu.make_async_copy(srcs, srcb, isem).wait()
    pltpu.make_async_copy(dsts, dstb, isem).wait()
    pltpu.make_async_copy(ws, wb, isem).wait()


def _spmm_pipeline(h_hbm, acc, rows, srcb, dstb, wb, gsem, ssem, nchunks):
    def _fire_gather(k, b):
        pltpu.async_copy(h_hbm.at[srcb.at[k]], rows.at[b], gsem.at[b])

    def _wait_gather(k, b):
        pltpu.make_async_copy(h_hbm.at[srcb.at[k]], rows.at[b],
                              gsem.at[b]).wait()

    def _fire_scatter(k, b):
        pltpu.async_copy(rows.at[b], acc.at[dstb.at[k]], ssem.at[b], add=True)

    def _wait_scatter(k, b):
        pltpu.make_async_copy(rows.at[b], acc.at[dstb.at[k]],
                              ssem.at[b]).wait()

    _fire_gather(0, 0)

    def chunk_body(k, carry):
        b = lax.rem(k, 2)
        nb = 1 - b
        _wait_gather(k, b)

        @pl.when(k >= 1)
        def _wait_prev_scatter():
            _wait_scatter(k - 1, nb)

        @pl.when(k + 1 < nchunks)
        def _fire_next_gather():
            _fire_gather(k + 1, nb)

        _scale_rows(rows, b, wb, k, rows.shape[2])
        _fire_scatter(k, b)
        return carry

    lax.fori_loop(0, nchunks, chunk_body, 0)
    _wait_scatter(nchunks - 1, (nchunks - 1) % 2)


@functools.partial(
    pl.kernel,
    out_type=jax.ShapeDtypeStruct((4, N, NFEAT), jnp.float32),
    mesh=_MESH,
    scratch_types=[
        pltpu.VMEM_SHARED((N, NFEAT), jnp.float32),
        pltpu.VMEM((2, C, NFEAT), jnp.float32),
        pltpu.VMEM((SK, C), jnp.int32),
        pltpu.VMEM((SK, C), jnp.int32),
        pltpu.VMEM((SK, C), jnp.float32),
        pltpu.SemaphoreType.DMA((2,)),
        pltpu.SemaphoreType.DMA((2,)),
        pltpu.SemaphoreType.DMA,
    ],
)
def _spmm4_sc(h_hbm, src_hbm, dst_hbm, w_hbm, zeros_hbm, out_hbm,
              acc, rows, srcb, dstb, wb, gsem, ssem, isem):
    cid = lax.axis_index("c")
    sid = lax.axis_index("s")
    row0 = sid * RPT
    crow0 = sid * KB

    for m_local in range(2):
        m = cid * 2 + m_local

        pltpu.sync_copy(zeros_hbm.at[pl.ds(row0, RPT), :],
                        acc.at[pl.ds(row0, RPT), :])

        @pl.when(sid == NTILE - 1)
        def _zero_tail():
            pltpu.sync_copy(zeros_hbm.at[pl.ds(RPT * NTILE, TAILN), :],
                            acc.at[pl.ds(RPT * NTILE, TAILN), :])

        plsc.subcore_barrier()

        for s in range(KB // SK):
            c0 = crow0 + s * SK
            _preload_block(src_hbm.at[m, pl.ds(c0, SK), :],
                           dst_hbm.at[m, pl.ds(c0, SK), :],
                           w_hbm.at[m, pl.ds(c0, SK), :],
                           srcb, dstb, wb, isem)
            _spmm_pipeline(h_hbm, acc, rows, srcb, dstb, wb, gsem, ssem, SK)

        plsc.subcore_barrier()

        pltpu.sync_copy(acc.at[pl.ds(row0, RPT), :],
                        out_hbm.at[m, pl.ds(row0, RPT), :])

        @pl.when(sid == NTILE - 1)
        def _out_tail():
            pltpu.sync_copy(acc.at[pl.ds(RPT * NTILE, TAILN), :],
                            out_hbm.at[m, pl.ds(RPT * NTILE, TAILN), :])

        plsc.subcore_barrier()


@functools.partial(
    pl.kernel,
    out_type=jax.ShapeDtypeStruct((NSC, N, NFEAT), jnp.float32),
    mesh=_MESH,
    scratch_types=[
        pltpu.VMEM_SHARED((N, NFEAT), jnp.float32),
        pltpu.VMEM((2, C, NFEAT), jnp.float32),
        pltpu.VMEM((SK, C), jnp.int32),
        pltpu.VMEM((SK, C), jnp.int32),
        pltpu.VMEM((SK, C), jnp.float32),
        pltpu.SemaphoreType.DMA((2,)),
        pltpu.SemaphoreType.DMA((2,)),
        pltpu.SemaphoreType.DMA,
    ],
)
def _spmm_adj_sc(sup_hbm, src_hbm, dst_hbm, w_hbm, zeros_hbm, out_hbm,
                 acc, rows, srcb, dstb, wb, gsem, ssem, isem):
    cid = lax.axis_index("c")
    sid = lax.axis_index("s")
    wid = sid * NSC + cid
    row0 = sid * RPT
    crow0 = wid * KD

    pltpu.sync_copy(zeros_hbm.at[pl.ds(row0, RPT), :],
                    acc.at[pl.ds(row0, RPT), :])

    @pl.when(sid == NTILE - 1)
    def _zero_tail():
        pltpu.sync_copy(zeros_hbm.at[pl.ds(RPT * NTILE, TAILN), :],
                        acc.at[pl.ds(RPT * NTILE, TAILN), :])

    plsc.subcore_barrier()

    for s in range(KD // SK):
        c0 = crow0 + s * SK
        _preload_block(src_hbm.at[pl.ds(c0, SK), :],
                       dst_hbm.at[pl.ds(c0, SK), :],
                       w_hbm.at[pl.ds(c0, SK), :],
                       srcb, dstb, wb, isem)
        _spmm_pipeline(sup_hbm, acc, rows, srcb, dstb, wb, gsem, ssem, SK)

    plsc.subcore_barrier()

    pltpu.sync_copy(acc.at[pl.ds(row0, RPT), :],
                    out_hbm.at[cid, pl.ds(row0, RPT), :])

    @pl.when(sid == NTILE - 1)
    def _out_tail():
        pltpu.sync_copy(acc.at[pl.ds(RPT * NTILE, TAILN), :],
                        out_hbm.at[cid, pl.ds(RPT * NTILE, TAILN), :])


_BN = 2000


def _mm_body(x_ref, w_ref, o_ref):
    o_ref[:, :] = jnp.dot(x_ref[:, :], w_ref[:, :],
                          preferred_element_type=jnp.float32)


def _attn_body(c0r, c1r, c2r, c3r, a_ref, r_ref, xcat_ref):
    chans = (c0r[0], jnp.abs(c1r[0]), jnp.abs(c2r[0]), jnp.abs(c3r[0]))
    es = []
    for c in range(4):
        e = jnp.dot(chans[c], a_ref[c], preferred_element_type=jnp.float32)
        es.append(jnp.where(e > 0, e, 0.2 * e))
    mx = jnp.maximum(jnp.maximum(es[0], es[1]), jnp.maximum(es[2], es[3]))
    ex = [jnp.exp(e - mx) for e in es]
    denom = ex[0] + ex[1] + ex[2] + ex[3]
    out = jnp.zeros_like(chans[0])
    for c in range(4):
        attn = ex[c] / denom
        out = out + jnp.dot(attn, r_ref[:, :],
                            preferred_element_type=jnp.float32) * chans[c]
    xcat_ref[:, :] = jnp.maximum(out, 0.0)


def _final_body(p0r, p1r, xcat_ref, wg_ref, bg_ref, o_ref):
    z = p0r[0] + p1r[0] + SMOO * xcat_ref[:, :]
    logits = (jnp.dot(z, wg_ref[:, :], preferred_element_type=jnp.float32)
              / (1.0 + SMOO)) + bg_ref[0, :][None, :]
    mx = jnp.max(logits, axis=1, keepdims=True)
    lse = mx + jnp.log(jnp.sum(jnp.exp(logits - mx), axis=1, keepdims=True))
    o_ref[:, :] = logits - lse


def _pad_chunks(arr, epad, nrows):
    return jnp.pad(arr, (0, epad - E)).reshape(nrows, C)


def kernel(x, A_tilde_index, A_tilde_weight, s1_sct_index, s1_sct_weight,
           s2_sct_index, s2_sct_weight, s3_sct_index, s3_sct_weight,
           adj_p_index, adj_p_weight, W, a, Wg, bg):
    f32 = jnp.float32

    W_all = W.transpose(1, 0, 2).reshape(NFEAT, NHEADS * HID)
    nrb = NTILE * KB
    src4 = jnp.stack([_pad_chunks(A_tilde_index[0], EPB, nrb),
                      _pad_chunks(s1_sct_index[0], EPB, nrb),
                      _pad_chunks(s2_sct_index[0], EPB, nrb),
                      _pad_chunks(s3_sct_index[0], EPB, nrb)])
    dst4 = jnp.stack([_pad_chunks(A_tilde_index[1], EPB, nrb),
                      _pad_chunks(s1_sct_index[1], EPB, nrb),
                      _pad_chunks(s2_sct_index[1], EPB, nrb),
                      _pad_chunks(s3_sct_index[1], EPB, nrb)])
    w4 = jnp.stack([_pad_chunks(A_tilde_weight, EPB, nrb),
                    _pad_chunks(s1_sct_weight, EPB, nrb),
                    _pad_chunks(s2_sct_weight, EPB, nrb),
                    _pad_chunks(s3_sct_weight, EPB, nrb)])
    nrd = NSC * NTILE * KD
    srca = _pad_chunks(adj_p_index[0], EPD, nrd)
    dsta = _pad_chunks(adj_p_index[1], EPD, nrd)
    wa = _pad_chunks(adj_p_weight, EPD, nrd)
    amat = (a.transpose(1, 0, 2)[:, :, :, None]
            * jnp.eye(NHEADS, dtype=f32)[None, :, None, :])
    amat = amat.reshape(4, NHEADS * HID, NHEADS)
    rmat = jnp.repeat(jnp.eye(NHEADS, dtype=f32), HID, axis=1)
    wg_pad = jnp.pad(Wg, ((0, 0), (0, DPAD - NCLASS)))
    bg_row = jnp.concatenate(
        [bg, jnp.full((DPAD - NCLASS,), -jnp.inf, dtype=f32)]).reshape(1, DPAD)
    z128 = jnp.zeros((N, NFEAT), dtype=f32)

    h = pl.pallas_call(
        _mm_body,
        grid=(N // _BN,),
        in_specs=[pl.BlockSpec((_BN, NFEAT), lambda i: (i, 0)),
                  pl.BlockSpec((NFEAT, NFEAT), lambda i: (0, 0))],
        out_specs=pl.BlockSpec((_BN, NFEAT), lambda i: (i, 0)),
        out_shape=jax.ShapeDtypeStruct((N, NFEAT), f32),
    )(x, W_all)

    c4 = _spmm4_sc(h, src4, dst4, w4, z128)

    def _csel(m):
        return pl.BlockSpec((1, _BN, NFEAT), lambda i, m=m: (m, i, 0))

    xcat = pl.pallas_call(
        _attn_body,
        grid=(N // _BN,),
        in_specs=[_csel(0), _csel(1), _csel(2), _csel(3),
                  pl.BlockSpec((4, NHEADS * HID, NHEADS), lambda i: (0, 0, 0)),
                  pl.BlockSpec((NHEADS, NHEADS * HID), lambda i: (0, 0))],
        out_specs=pl.BlockSpec((_BN, NFEAT), lambda i: (i, 0)),
        out_shape=jax.ShapeDtypeStruct((N, NFEAT), f32),
    )(c4, c4, c4, c4, amat, rmat)

    p2 = _spmm_adj_sc(xcat, srca, dsta, wa, z128)

    out16 = pl.pallas_call(
        _final_body,
        grid=(N // _BN,),
        in_specs=[pl.BlockSpec((1, _BN, NFEAT), lambda i: (0, i, 0)),
                  pl.BlockSpec((1, _BN, NFEAT), lambda i: (1, i, 0)),
                  pl.BlockSpec((_BN, NFEAT), lambda i: (i, 0)),
                  pl.BlockSpec((NHEADS * HID, DPAD), lambda i: (0, 0)),
                  pl.BlockSpec((1, DPAD), lambda i: (0, 0))],
        out_specs=pl.BlockSpec((_BN, DPAD), lambda i: (i, 0)),
        out_shape=jax.ShapeDtypeStruct((N, DPAD), f32),
    )(p2, p2, xcat, wg_pad, bg_row)

    return out16[:, :NCLASS]


# speedup vs baseline: 1.2994x; 1.2994x over previous
import functools

import jax
import jax.numpy as jnp
from jax import lax
from jax.experimental import pallas as pl
from jax.experimental.pallas import tpu as pltpu
from jax.experimental.pallas import tpu_sc as plsc

N = 10000
E = 320000
NFEAT = 128
HID = 16
NHEADS = 8
NCLASS = 10
SMOO = 0.5

C = 128
NTILE = 16
NSC = 2
RPT = 624
TAILN = N - RPT * NTILE
DPAD = 16

KB = 160
EPB = NTILE * KB * C
KD = 80
EPD = NSC * NTILE * KD * C

_MESH = plsc.VectorSubcoreMesh(core_axis_name="c", subcore_axis_name="s")


def _scale_rows(rows, wrow, k, nfeat):
    def group_body(g, carry):
        e0 = g * 16
        wv = wrow[k, pl.ds(e0, 16)]
        for j in range(16):
            w = wv[j]
            for f in range(nfeat // 16):
                sl = pl.ds(f * 16, 16)
                rows[e0 + j, sl] = rows[e0 + j, sl] * w
        return carry

    lax.fori_loop(0, C // 16, group_body, 0)


SK = 40


def _preload_block(srcs, dsts, ws, srcb, dstb, wb, isem):
    pltpu.async_copy(srcs, srcb, isem)
    pltpu.async_copy(dsts, dstb, isem)
    pltpu.async_copy(ws, wb, isem)
    pltpu.make_async_copy(srcs, srcb, isem).wait()
    pltpu.make_async_copy(dsts, dstb, isem).wait()
    pltpu.make_async_copy(ws, wb, isem).wait()


def _spmm_pipeline(h_hbm, acc, bufs, srcb, dstb, wb, nchunks):
    def _fire_gather(k, b):
        pltpu.async_copy(h_hbm.at[srcb.at[k]], bufs[b][0], bufs[b][1])

    def _wait_gather(k, b):
        pltpu.make_async_copy(h_hbm.at[srcb.at[k]], bufs[b][0],
                              bufs[b][1]).wait()

    def _fire_scatter(k, b):
        pltpu.async_copy(bufs[b][0], acc.at[dstb.at[k]], bufs[b][2], add=True)

    def _wait_scatter(k, b):
        pltpu.make_async_copy(bufs[b][0], acc.at[dstb.at[k]],
                              bufs[b][2]).wait()

    _fire_gather(0, 0)

    def pair_body(j, carry):
        k = j * 2
        _wait_gather(k, 0)
        _scale_rows(bufs[0][0], wb, k, bufs[0][0].shape[1])

        @pl.when(j == 0)
        def _prime():
            _fire_gather(1, 1)

        @pl.when(j >= 1)
        def _steady():
            _wait_scatter(k - 1, 1)
            _fire_gather(k + 1, 1)

        _fire_scatter(k, 0)

        _wait_gather(k + 1, 1)
        _scale_rows(bufs[1][0], wb, k + 1, bufs[1][0].shape[1])
        _wait_scatter(k, 0)

        @pl.when(j + 1 < nchunks // 2)
        def _next():
            _fire_gather(k + 2, 0)

        _fire_scatter(k + 1, 1)
        return carry

    lax.fori_loop(0, nchunks // 2, pair_body, 0)
    _wait_scatter(nchunks - 1, 1)


@functools.partial(
    pl.kernel,
    out_type=jax.ShapeDtypeStruct((4, N, NFEAT), jnp.float32),
    mesh=_MESH,
    scratch_types=[
        pltpu.VMEM_SHARED((N, NFEAT), jnp.float32),
        pltpu.VMEM((C, NFEAT), jnp.float32),
        pltpu.VMEM((C, NFEAT), jnp.float32),
        pltpu.VMEM((SK, C), jnp.int32),
        pltpu.VMEM((SK, C), jnp.int32),
        pltpu.VMEM((SK, C), jnp.float32),
        pltpu.SemaphoreType.DMA,
        pltpu.SemaphoreType.DMA,
        pltpu.SemaphoreType.DMA,
        pltpu.SemaphoreType.DMA,
        pltpu.SemaphoreType.DMA,
    ],
)
def _spmm4_sc(h_hbm, src_hbm, dst_hbm, w_hbm, zeros_hbm, out_hbm,
              acc, rows0, rows1, srcb, dstb, wb, gs0, gs1, ss0, ss1, isem):
    bufs = ((rows0, gs0, ss0), (rows1, gs1, ss1))
    cid = lax.axis_index("c")
    sid = lax.axis_index("s")
    row0 = sid * RPT
    crow0 = sid * KB

    for m_local in range(2):
        m = cid * 2 + m_local

        pltpu.sync_copy(zeros_hbm.at[pl.ds(row0, RPT), :],
                        acc.at[pl.ds(row0, RPT), :])

        @pl.when(sid == NTILE - 1)
        def _zero_tail():
            pltpu.sync_copy(zeros_hbm.at[pl.ds(RPT * NTILE, TAILN), :],
                            acc.at[pl.ds(RPT * NTILE, TAILN), :])

        plsc.subcore_barrier()

        for s in range(KB // SK):
            c0 = crow0 + s * SK
            _preload_block(src_hbm.at[m, pl.ds(c0, SK), :],
                           dst_hbm.at[m, pl.ds(c0, SK), :],
                           w_hbm.at[m, pl.ds(c0, SK), :],
                           srcb, dstb, wb, isem)
            _spmm_pipeline(h_hbm, acc, bufs, srcb, dstb, wb, SK)

        plsc.subcore_barrier()

        pltpu.sync_copy(acc.at[pl.ds(row0, RPT), :],
                        out_hbm.at[m, pl.ds(row0, RPT), :])

        @pl.when(sid == NTILE - 1)
        def _out_tail():
            pltpu.sync_copy(acc.at[pl.ds(RPT * NTILE, TAILN), :],
                            out_hbm.at[m, pl.ds(RPT * NTILE, TAILN), :])

        plsc.subcore_barrier()


@functools.partial(
    pl.kernel,
    out_type=jax.ShapeDtypeStruct((NSC, N, NFEAT), jnp.float32),
    mesh=_MESH,
    scratch_types=[
        pltpu.VMEM_SHARED((N, NFEAT), jnp.float32),
        pltpu.VMEM((C, NFEAT), jnp.float32),
        pltpu.VMEM((C, NFEAT), jnp.float32),
        pltpu.VMEM((SK, C), jnp.int32),
        pltpu.VMEM((SK, C), jnp.int32),
        pltpu.VMEM((SK, C), jnp.float32),
        pltpu.SemaphoreType.DMA,
        pltpu.SemaphoreType.DMA,
        pltpu.SemaphoreType.DMA,
        pltpu.SemaphoreType.DMA,
        pltpu.SemaphoreType.DMA,
    ],
)
def _spmm_adj_sc(sup_hbm, src_hbm, dst_hbm, w_hbm, zeros_hbm, out_hbm,
                 acc, rows0, rows1, srcb, dstb, wb, gs0, gs1, ss0, ss1, isem):
    bufs = ((rows0, gs0, ss0), (rows1, gs1, ss1))
    cid = lax.axis_index("c")
    sid = lax.axis_index("s")
    wid = sid * NSC + cid
    row0 = sid * RPT
    crow0 = wid * KD

    pltpu.sync_copy(zeros_hbm.at[pl.ds(row0, RPT), :],
                    acc.at[pl.ds(row0, RPT), :])

    @pl.when(sid == NTILE - 1)
    def _zero_tail():
        pltpu.sync_copy(zeros_hbm.at[pl.ds(RPT * NTILE, TAILN), :],
                        acc.at[pl.ds(RPT * NTILE, TAILN), :])

    plsc.subcore_barrier()

    for s in range(KD // SK):
        c0 = crow0 + s * SK
        _preload_block(src_hbm.at[pl.ds(c0, SK), :],
                       dst_hbm.at[pl.ds(c0, SK), :],
                       w_hbm.at[pl.ds(c0, SK), :],
                       srcb, dstb, wb, isem)
        _spmm_pipeline(sup_hbm, acc, bufs, srcb, dstb, wb, SK)

    plsc.subcore_barrier()

    pltpu.sync_copy(acc.at[pl.ds(row0, RPT), :],
                    out_hbm.at[cid, pl.ds(row0, RPT), :])

    @pl.when(sid == NTILE - 1)
    def _out_tail():
        pltpu.sync_copy(acc.at[pl.ds(RPT * NTILE, TAILN), :],
                        out_hbm.at[cid, pl.ds(RPT * NTILE, TAILN), :])


_BN = 2000


def _mm_body(x_ref, w_ref, o_ref):
    o_ref[:, :] = jnp.dot(x_ref[:, :], w_ref[:, :],
                          preferred_element_type=jnp.float32)


def _attn_body(c0r, c1r, c2r, c3r, a_ref, r_ref, xcat_ref):
    chans = (c0r[0], jnp.abs(c1r[0]), jnp.abs(c2r[0]), jnp.abs(c3r[0]))
    es = []
    for c in range(4):
        e = jnp.dot(chans[c], a_ref[c], preferred_element_type=jnp.float32)
        es.append(jnp.where(e > 0, e, 0.2 * e))
    mx = jnp.maximum(jnp.maximum(es[0], es[1]), jnp.maximum(es[2], es[3]))
    ex = [jnp.exp(e - mx) for e in es]
    denom = ex[0] + ex[1] + ex[2] + ex[3]
    out = jnp.zeros_like(chans[0])
    for c in range(4):
        attn = ex[c] / denom
        out = out + jnp.dot(attn, r_ref[:, :],
                            preferred_element_type=jnp.float32) * chans[c]
    xcat_ref[:, :] = jnp.maximum(out, 0.0)


def _final_body(p0r, p1r, xcat_ref, wg_ref, bg_ref, o_ref):
    z = p0r[0] + p1r[0] + SMOO * xcat_ref[:, :]
    logits = (jnp.dot(z, wg_ref[:, :], preferred_element_type=jnp.float32)
              / (1.0 + SMOO)) + bg_ref[0, :][None, :]
    mx = jnp.max(logits, axis=1, keepdims=True)
    lse = mx + jnp.log(jnp.sum(jnp.exp(logits - mx), axis=1, keepdims=True))
    o_ref[:, :] = logits - lse


def _pad_chunks(arr, epad, nrows):
    return jnp.pad(arr, (0, epad - E)).reshape(nrows, C)


def kernel(x, A_tilde_index, A_tilde_weight, s1_sct_index, s1_sct_weight,
           s2_sct_index, s2_sct_weight, s3_sct_index, s3_sct_weight,
           adj_p_index, adj_p_weight, W, a, Wg, bg):
    f32 = jnp.float32

    W_all = W.transpose(1, 0, 2).reshape(NFEAT, NHEADS * HID)
    nrb = NTILE * KB
    src4 = jnp.stack([_pad_chunks(A_tilde_index[0], EPB, nrb),
                      _pad_chunks(s1_sct_index[0], EPB, nrb),
                      _pad_chunks(s2_sct_index[0], EPB, nrb),
                      _pad_chunks(s3_sct_index[0], EPB, nrb)])
    dst4 = jnp.stack([_pad_chunks(A_tilde_index[1], EPB, nrb),
                      _pad_chunks(s1_sct_index[1], EPB, nrb),
                      _pad_chunks(s2_sct_index[1], EPB, nrb),
                      _pad_chunks(s3_sct_index[1], EPB, nrb)])
    w4 = jnp.stack([_pad_chunks(A_tilde_weight, EPB, nrb),
                    _pad_chunks(s1_sct_weight, EPB, nrb),
                    _pad_chunks(s2_sct_weight, EPB, nrb),
                    _pad_chunks(s3_sct_weight, EPB, nrb)])
    nrd = NSC * NTILE * KD
    srca = _pad_chunks(adj_p_index[0], EPD, nrd)
    dsta = _pad_chunks(adj_p_index[1], EPD, nrd)
    wa = _pad_chunks(adj_p_weight, EPD, nrd)
    amat = (a.transpose(1, 0, 2)[:, :, :, None]
            * jnp.eye(NHEADS, dtype=f32)[None, :, None, :])
    amat = amat.reshape(4, NHEADS * HID, NHEADS)
    rmat = jnp.repeat(jnp.eye(NHEADS, dtype=f32), HID, axis=1)
    wg_pad = jnp.pad(Wg, ((0, 0), (0, DPAD - NCLASS)))
    bg_row = jnp.concatenate(
        [bg, jnp.full((DPAD - NCLASS,), -jnp.inf, dtype=f32)]).reshape(1, DPAD)
    z128 = jnp.zeros((N, NFEAT), dtype=f32)

    h = pl.pallas_call(
        _mm_body,
        grid=(N // _BN,),
        in_specs=[pl.BlockSpec((_BN, NFEAT), lambda i: (i, 0)),
                  pl.BlockSpec((NFEAT, NFEAT), lambda i: (0, 0))],
        out_specs=pl.BlockSpec((_BN, NFEAT), lambda i: (i, 0)),
        out_shape=jax.ShapeDtypeStruct((N, NFEAT), f32),
    )(x, W_all)

    c4 = _spmm4_sc(h, src4, dst4, w4, z128)

    def _csel(m):
        return pl.BlockSpec((1, _BN, NFEAT), lambda i, m=m: (m, i, 0))

    xcat = pl.pallas_call(
        _attn_body,
        grid=(N // _BN,),
        in_specs=[_csel(0), _csel(1), _csel(2), _csel(3),
                  pl.BlockSpec((4, NHEADS * HID, NHEADS), lambda i: (0, 0, 0)),
                  pl.BlockSpec((NHEADS, NHEADS * HID), lambda i: (0, 0))],
        out_specs=pl.BlockSpec((_BN, NFEAT), lambda i: (i, 0)),
        out_shape=jax.ShapeDtypeStruct((N, NFEAT), f32),
    )(c4, c4, c4, c4, amat, rmat)

    p2 = _spmm_adj_sc(xcat, srca, dsta, wa, z128)

    out16 = pl.pallas_call(
        _final_body,
        grid=(N // _BN,),
        in_specs=[pl.BlockSpec((1, _BN, NFEAT), lambda i: (0, i, 0)),
                  pl.BlockSpec((1, _BN, NFEAT), lambda i: (1, i, 0)),
                  pl.BlockSpec((_BN, NFEAT), lambda i: (i, 0)),
                  pl.BlockSpec((NHEADS * HID, DPAD), lambda i: (0, 0)),
                  pl.BlockSpec((1, DPAD), lambda i: (0, 0))],
        out_specs=pl.BlockSpec((_BN, DPAD), lambda i: (i, 0)),
        out_shape=jax.ShapeDtypeStruct((N, DPAD), f32),
    )(p2, p2, xcat, wg_pad, bg_row)

    return out16[:, :NCLASS]


# speedup vs baseline: 1.3527x; 1.0410x over previous
import functools

import jax
import jax.numpy as jnp
from jax import lax
from jax.experimental import pallas as pl
from jax.experimental.pallas import tpu as pltpu
from jax.experimental.pallas import tpu_sc as plsc

N = 10000
E = 320000
NFEAT = 128
HID = 16
NHEADS = 8
NCLASS = 10
SMOO = 0.5

C = 128
NTILE = 16
NSC = 2
RPT = 624
TAILN = N - RPT * NTILE
DPAD = 16

KB = 160
EPB = NTILE * KB * C
KD = 80
EPD = NSC * NTILE * KD * C
RBUF = 2
IBUF = 4

_MESH = plsc.VectorSubcoreMesh(core_axis_name="c", subcore_axis_name="s")


def _scale_rows(rows, wb, nfeat):
    def group_body(g, carry):
        e0 = g * 16
        wv = wb[pl.ds(e0, 16)]
        for j in range(16):
            w = wv[j]
            for f in range(nfeat // 16):
                sl = pl.ds(f * 16, 16)
                rows[e0 + j, sl] = rows[e0 + j, sl] * w
        return carry

    lax.fori_loop(0, C // 16, group_body, 0)


def _spmm_pipeline(h_hbm, acc, bufs, slicers, nchunks):
    rows, gsem, ssem, srcb, dstb, wb, isem = bufs

    def _fire_idx(k, s):
        pltpu.async_copy(slicers[0](k), srcb[s], isem[s])
        pltpu.async_copy(slicers[1](k), dstb[s], isem[s])
        pltpu.async_copy(slicers[2](k), wb[s], isem[s])

    def _wait_idx(k, s):
        pltpu.make_async_copy(slicers[0](k), srcb[s], isem[s]).wait()
        pltpu.make_async_copy(slicers[1](k), dstb[s], isem[s]).wait()
        pltpu.make_async_copy(slicers[2](k), wb[s], isem[s]).wait()

    def _fire_gather(rb, s):
        pltpu.async_copy(h_hbm.at[srcb[s]], rows[rb], gsem[rb])

    def _wait_gather(rb, s):
        pltpu.make_async_copy(h_hbm.at[srcb[s]], rows[rb], gsem[rb]).wait()

    def _fire_scatter(rb, s):
        pltpu.async_copy(rows[rb], acc.at[dstb[s]], ssem[rb], add=True)

    def _wait_scatter(rb, s):
        pltpu.make_async_copy(rows[rb], acc.at[dstb[s]], ssem[rb]).wait()

    def _chunk(k, s, do_ws, do_g, do_i):
        rb = s % RBUF
        _wait_gather(rb, s)
        if do_ws:
            _wait_scatter((rb + 1) % RBUF, (s + 3) % IBUF)
        if do_g:
            _wait_idx(k + 1, (s + 1) % IBUF)
            _fire_gather((rb + 1) % RBUF, (s + 1) % IBUF)
        _scale_rows(rows[rb], wb[s], rows[rb].shape[1])
        _fire_scatter(rb, s)
        if do_i:
            _fire_idx(k + 2, (s + 2) % IBUF)

    _fire_idx(0, 0)
    _fire_idx(1, 1)
    _wait_idx(0, 0)
    _fire_gather(0, 0)

    _chunk(0, 0, False, True, True)
    for r in range(1, IBUF):
        _chunk(r, r, True, True, True)

    def quad_body(j, carry):
        for r in range(IBUF):
            _chunk(j * IBUF + r, r, True, True, True)
        return carry

    lax.fori_loop(1, nchunks // IBUF - 1, quad_body, 0)

    k0 = nchunks - IBUF
    _chunk(k0, k0 % IBUF, True, True, True)
    _chunk(k0 + 1, (k0 + 1) % IBUF, True, True, True)
    _chunk(k0 + 2, (k0 + 2) % IBUF, True, True, False)
    _chunk(k0 + 3, (k0 + 3) % IBUF, True, False, False)
    _wait_scatter((k0 + 3) % RBUF, (k0 + 3) % IBUF)


_RING_SCRATCH = (
    [pltpu.VMEM((C, NFEAT), jnp.float32) for _ in range(RBUF)]
    + [pltpu.SemaphoreType.DMA for _ in range(2 * RBUF)]
    + [pltpu.VMEM((C,), jnp.int32) for _ in range(IBUF)]
    + [pltpu.VMEM((C,), jnp.int32) for _ in range(IBUF)]
    + [pltpu.VMEM((C,), jnp.float32) for _ in range(IBUF)]
    + [pltpu.SemaphoreType.DMA for _ in range(IBUF)]
)


def _mk_bufs(scr):
    rows = scr[0:RBUF]
    gsem = scr[RBUF:2 * RBUF]
    ssem = scr[2 * RBUF:3 * RBUF]
    o = 3 * RBUF
    srcb = scr[o:o + IBUF]
    dstb = scr[o + IBUF:o + 2 * IBUF]
    wb = scr[o + 2 * IBUF:o + 3 * IBUF]
    isem = scr[o + 3 * IBUF:o + 4 * IBUF]
    return (rows, gsem, ssem, srcb, dstb, wb, isem)


@functools.partial(
    pl.kernel,
    out_type=jax.ShapeDtypeStruct((4, N, NFEAT), jnp.float32),
    mesh=_MESH,
    scratch_types=[pltpu.VMEM_SHARED((N, NFEAT), jnp.float32)]
    + _RING_SCRATCH,
)
def _spmm4_sc(h_hbm, src_hbm, dst_hbm, w_hbm, zeros_hbm, out_hbm,
              acc, *scr):
    bufs = _mk_bufs(scr)
    cid = lax.axis_index("c")
    sid = lax.axis_index("s")
    row0 = sid * RPT
    e0 = sid * KB * C

    for m_local in range(2):
        m = cid * 2 + m_local
        slicers = (lambda k: src_hbm.at[m, pl.ds(e0 + k * C, C)],
                   lambda k: dst_hbm.at[m, pl.ds(e0 + k * C, C)],
                   lambda k: w_hbm.at[m, pl.ds(e0 + k * C, C)])

        pltpu.sync_copy(zeros_hbm.at[pl.ds(row0, RPT), :],
                        acc.at[pl.ds(row0, RPT), :])

        @pl.when(sid == NTILE - 1)
        def _zero_tail():
            pltpu.sync_copy(zeros_hbm.at[pl.ds(RPT * NTILE, TAILN), :],
                            acc.at[pl.ds(RPT * NTILE, TAILN), :])

        plsc.subcore_barrier()

        _spmm_pipeline(h_hbm, acc, bufs, slicers, KB)

        plsc.subcore_barrier()

        pltpu.sync_copy(acc.at[pl.ds(row0, RPT), :],
                        out_hbm.at[m, pl.ds(row0, RPT), :])

        @pl.when(sid == NTILE - 1)
        def _out_tail():
            pltpu.sync_copy(acc.at[pl.ds(RPT * NTILE, TAILN), :],
                            out_hbm.at[m, pl.ds(RPT * NTILE, TAILN), :])

        plsc.subcore_barrier()


@functools.partial(
    pl.kernel,
    out_type=jax.ShapeDtypeStruct((NSC, N, NFEAT), jnp.float32),
    mesh=_MESH,
    scratch_types=[pltpu.VMEM_SHARED((N, NFEAT), jnp.float32)]
    + _RING_SCRATCH,
)
def _spmm_adj_sc(sup_hbm, src_hbm, dst_hbm, w_hbm, zeros_hbm, out_hbm,
                 acc, *scr):
    bufs = _mk_bufs(scr)
    cid = lax.axis_index("c")
    sid = lax.axis_index("s")
    wid = sid * NSC + cid
    row0 = sid * RPT
    e0 = wid * KD * C
    slicers = (lambda k: src_hbm.at[pl.ds(e0 + k * C, C)],
               lambda k: dst_hbm.at[pl.ds(e0 + k * C, C)],
               lambda k: w_hbm.at[pl.ds(e0 + k * C, C)])

    pltpu.sync_copy(zeros_hbm.at[pl.ds(row0, RPT), :],
                    acc.at[pl.ds(row0, RPT), :])

    @pl.when(sid == NTILE - 1)
    def _zero_tail():
        pltpu.sync_copy(zeros_hbm.at[pl.ds(RPT * NTILE, TAILN), :],
                        acc.at[pl.ds(RPT * NTILE, TAILN), :])

    plsc.subcore_barrier()

    _spmm_pipeline(sup_hbm, acc, bufs, slicers, KD)

    plsc.subcore_barrier()

    pltpu.sync_copy(acc.at[pl.ds(row0, RPT), :],
                    out_hbm.at[cid, pl.ds(row0, RPT), :])

    @pl.when(sid == NTILE - 1)
    def _out_tail():
        pltpu.sync_copy(acc.at[pl.ds(RPT * NTILE, TAILN), :],
                        out_hbm.at[cid, pl.ds(RPT * NTILE, TAILN), :])


_BN = 2000


def _mm_body(x_ref, w_ref, o_ref):
    o_ref[:, :] = jnp.dot(x_ref[:, :], w_ref[:, :],
                          preferred_element_type=jnp.float32)


def _attn_body(c0r, c1r, c2r, c3r, a_ref, r_ref, xcat_ref):
    chans = (c0r[0], jnp.abs(c1r[0]), jnp.abs(c2r[0]), jnp.abs(c3r[0]))
    es = []
    for c in range(4):
        e = jnp.dot(chans[c], a_ref[c], preferred_element_type=jnp.float32)
        es.append(jnp.where(e > 0, e, 0.2 * e))
    mx = jnp.maximum(jnp.maximum(es[0], es[1]), jnp.maximum(es[2], es[3]))
    ex = [jnp.exp(e - mx) for e in es]
    denom = ex[0] + ex[1] + ex[2] + ex[3]
    out = jnp.zeros_like(chans[0])
    for c in range(4):
        attn = ex[c] / denom
        out = out + jnp.dot(attn, r_ref[:, :],
                            preferred_element_type=jnp.float32) * chans[c]
    xcat_ref[:, :] = jnp.maximum(out, 0.0)


def _final_body(p0r, p1r, xcat_ref, wg_ref, bg_ref, o_ref):
    z = p0r[0] + p1r[0] + SMOO * xcat_ref[:, :]
    logits = (jnp.dot(z, wg_ref[:, :], preferred_element_type=jnp.float32)
              / (1.0 + SMOO)) + bg_ref[0, :][None, :]
    mx = jnp.max(logits, axis=1, keepdims=True)
    lse = mx + jnp.log(jnp.sum(jnp.exp(logits - mx), axis=1, keepdims=True))
    o_ref[:, :] = logits - lse


def _pad_chunks(arr, epad):
    return jnp.pad(arr, (0, epad - E))


def kernel(x, A_tilde_index, A_tilde_weight, s1_sct_index, s1_sct_weight,
           s2_sct_index, s2_sct_weight, s3_sct_index, s3_sct_weight,
           adj_p_index, adj_p_weight, W, a, Wg, bg):
    f32 = jnp.float32

    W_all = W.transpose(1, 0, 2).reshape(NFEAT, NHEADS * HID)
    src4 = jnp.stack([_pad_chunks(A_tilde_index[0], EPB),
                      _pad_chunks(s1_sct_index[0], EPB),
                      _pad_chunks(s2_sct_index[0], EPB),
                      _pad_chunks(s3_sct_index[0], EPB)])
    dst4 = jnp.stack([_pad_chunks(A_tilde_index[1], EPB),
                      _pad_chunks(s1_sct_index[1], EPB),
                      _pad_chunks(s2_sct_index[1], EPB),
                      _pad_chunks(s3_sct_index[1], EPB)])
    w4 = jnp.stack([_pad_chunks(A_tilde_weight, EPB),
                    _pad_chunks(s1_sct_weight, EPB),
                    _pad_chunks(s2_sct_weight, EPB),
                    _pad_chunks(s3_sct_weight, EPB)])
    srca = _pad_chunks(adj_p_index[0], EPD)
    dsta = _pad_chunks(adj_p_index[1], EPD)
    wa = _pad_chunks(adj_p_weight, EPD)
    amat = (a.transpose(1, 0, 2)[:, :, :, None]
            * jnp.eye(NHEADS, dtype=f32)[None, :, None, :])
    amat = amat.reshape(4, NHEADS * HID, NHEADS)
    rmat = jnp.repeat(jnp.eye(NHEADS, dtype=f32), HID, axis=1)
    wg_pad = jnp.pad(Wg, ((0, 0), (0, DPAD - NCLASS)))
    bg_row = jnp.concatenate(
        [bg, jnp.full((DPAD - NCLASS,), -jnp.inf, dtype=f32)]).reshape(1, DPAD)
    z128 = jnp.zeros((N, NFEAT), dtype=f32)

    h = pl.pallas_call(
        _mm_body,
        grid=(N // _BN,),
        in_specs=[pl.BlockSpec((_BN, NFEAT), lambda i: (i, 0)),
                  pl.BlockSpec((NFEAT, NFEAT), lambda i: (0, 0))],
        out_specs=pl.BlockSpec((_BN, NFEAT), lambda i: (i, 0)),
        out_shape=jax.ShapeDtypeStruct((N, NFEAT), f32),
    )(x, W_all)

    c4 = _spmm4_sc(h, src4, dst4, w4, z128)

    def _csel(m):
        return pl.BlockSpec((1, _BN, NFEAT), lambda i, m=m: (m, i, 0))

    xcat = pl.pallas_call(
        _attn_body,
        grid=(N // _BN,),
        in_specs=[_csel(0), _csel(1), _csel(2), _csel(3),
                  pl.BlockSpec((4, NHEADS * HID, NHEADS), lambda i: (0, 0, 0)),
                  pl.BlockSpec((NHEADS, NHEADS * HID), lambda i: (0, 0))],
        out_specs=pl.BlockSpec((_BN, NFEAT), lambda i: (i, 0)),
        out_shape=jax.ShapeDtypeStruct((N, NFEAT), f32),
    )(c4, c4, c4, c4, amat, rmat)

    p2 = _spmm_adj_sc(xcat, srca, dsta, wa, z128)

    out16 = pl.pallas_call(
        _final_body,
        grid=(N // _BN,),
        in_specs=[pl.BlockSpec((1, _BN, NFEAT), lambda i: (0, i, 0)),
                  pl.BlockSpec((1, _BN, NFEAT), lambda i: (1, i, 0)),
                  pl.BlockSpec((_BN, NFEAT), lambda i: (i, 0)),
                  pl.BlockSpec((NHEADS * HID, DPAD), lambda i: (0, 0)),
                  pl.BlockSpec((1, DPAD), lambda i: (0, 0))],
        out_specs=pl.BlockSpec((_BN, DPAD), lambda i: (i, 0)),
        out_shape=jax.ShapeDtypeStruct((N, DPAD), f32),
    )(p2, p2, xcat, wg_pad, bg_row)

    return out16[:, :NCLASS]
